# baseline (speedup 1.0000x reference)
import functools

import jax
import jax.numpy as jnp
from jax import lax
from jax.experimental import pallas as pl
from jax.experimental.pallas import tpu as pltpu
from jax.experimental.pallas import tpu_sc as plsc

N = 10000
E = 320000
D = 128
H = 128
NB = E // 128
NBH = NB // 2
NSUB = 16
NP = 10240
ROWS_A = NP // NSUB
RA = 632
F32 = jnp.float32
I32 = jnp.int32

_mesh = plsc.VectorSubcoreMesh(core_axis_name="c", subcore_axis_name="s",
                               num_cores=2, num_subcores=NSUB)


def _rsqrt16(x):
    b = lax.bitcast_convert_type(x, I32)
    y = lax.bitcast_convert_type(jnp.int32(0x5F3759DF) - (b >> 1), F32)
    for _ in range(3):
        y = y * (1.5 - 0.5 * x * y * y)
    return jnp.where(x > 0, y, 0.0)


def _rowshare(s, fn):
    @pl.when(s < 15)
    def _():
        for off, sz in ((0, 128), (128, 128), (256, 128), (384, 128),
                        (512, 120)):
            fn(s * RA + off, sz)

    @pl.when(s == 15)
    def _():
        for off, sz in ((0, 128), (128, 128), (256, 128), (384, 128),
                        (512, 8)):
            fn(15 * RA + off, sz)


def _f32(v):
    return lax.bitcast_convert_type(v, F32)


def _k_deg(erest, zeros1d, dis_out,
           ebuf, dstbuf, wbuf, dbuf, disbuf, dacc):
    c = lax.axis_index("c")
    s = lax.axis_index("s")
    cf = c.astype(F32)

    pltpu.sync_copy(zeros1d, dbuf)
    pltpu.sync_copy(dbuf, dacc.at[pl.ds(s * ROWS_A, ROWS_A)])
    plsc.subcore_barrier()

    def body(i, _):
        blk = i * NSUB + s

        @pl.when(blk < NB)
        def _():
            pltpu.sync_copy(erest.at[pl.ds(blk * 384, 384)], ebuf)
            for jj in range(8):
                sl = pl.ds(jj * 16, 16)
                e16 = _f32(ebuf[pl.ds(128 + jj * 16, 16)])
                m16 = _f32(ebuf[pl.ds(256 + jj * 16, 16)])
                wbuf[sl] = e16 * ((1.0 - cf) + cf * m16)
                dstbuf[sl] = ebuf[sl]
            pltpu.sync_copy(wbuf, dacc.at[dstbuf], add=True)
        return 0

    lax.fori_loop(0, (NB + NSUB - 1) // NSUB, body, 0)
    plsc.subcore_barrier()

    base = s * ROWS_A
    pltpu.sync_copy(dacc.at[pl.ds(base, ROWS_A)], dbuf)

    def rbody(j, _):
        sl = pl.ds(j * 16, 16)
        disbuf[sl] = _rsqrt16(dbuf[sl])
        return 0

    lax.fori_loop(0, ROWS_A // 16, rbody, 0)
    pltpu.sync_copy(disbuf, dis_out.at[pl.ds(c * NP + base, ROWS_A)])


@functools.partial(
    pl.kernel,
    out_type=jax.ShapeDtypeStruct((2 * NP,), F32),
    mesh=_mesh,
    scratch_types=[
        pltpu.VMEM((384,), I32),
        pltpu.VMEM((128,), I32),
        pltpu.VMEM((128,), F32),
        pltpu.VMEM((ROWS_A,), F32),
        pltpu.VMEM((ROWS_A,), F32),
        pltpu.VMEM_SHARED((NP,), F32),
    ],
)
def k_deg(*refs):
    _k_deg(*refs)


def _k_mat_body(x_ref, wp_ref, dis_ref, out_ref):
    hh = jnp.dot(x_ref[...], wp_ref[0], preferred_element_type=F32)
    out_ref[...] = hh * dis_ref[0]


def k_mat(x, wp, disr):
    return pl.pallas_call(
        _k_mat_body,
        grid=(2, 25),
        in_specs=[
            pl.BlockSpec((400, D), lambda c, r: (r, 0)),
            pl.BlockSpec((1, D, 128), lambda c, r: (c, 0, 0)),
            pl.BlockSpec((1, 400, 1), lambda c, r: (0, r, 0)),
        ],
        out_specs=pl.BlockSpec((400, 128), lambda c, r: (c * 25 + r, 0)),
        out_shape=jax.ShapeDtypeStruct((2 * N, 128), F32),
    )(x, wp, disr)


def _k_mat2_body(x_ref, w_ref, dis_ref, out_ref):
    hh = jnp.dot(x_ref[...], w_ref[...], preferred_element_type=F32)
    out_ref[...] = hh * dis_ref[0]


def k_mat2(x, W, disr):
    return pl.pallas_call(
        _k_mat2_body,
        grid=(25,),
        in_specs=[
            pl.BlockSpec((400, D), lambda r: (r, 0)),
            pl.BlockSpec((D, H), lambda r: (0, 0)),
            pl.BlockSpec((1, 400, 1), lambda r: (1, r, 0)),
        ],
        out_specs=pl.BlockSpec((400, 128), lambda r: (r, 0)),
        out_shape=jax.ShapeDtypeStruct((N, 128), F32),
    )(x, W, disr)


def _k_msga(hcat, esrc, erest, zeros_a, acc_out_a,
            sb0, sb1, eb0, eb1, dst0, dst1,
            rows0, rows1, msg_a,
            sem0, sem1, acc_as):
    c = lax.axis_index("c")
    s = lax.axis_index("s")
    sb = (sb0, sb1)
    eb = (eb0, eb1)
    dstb = (dst0, dst1)
    rows = (rows0, rows1)
    sem = (sem0, sem1)

    pltpu.sync_copy(zeros_a, msg_a)

    def _zero(off, sz):
        pltpu.sync_copy(msg_a.at[pl.ds(0, sz)], acc_as.at[pl.ds(off, sz)])

    _rowshare(s, _zero)
    plsc.subcore_barrier()

    coff = c * N
    NI = (NB + NSUB - 1) // NSUB

    def vb(i):
        return (i * NSUB + s) < NB

    def issue1(par, i):
        blk = i * NSUB + s

        @pl.when(vb(i))
        def _():
            pltpu.async_copy(esrc.at[pl.ds(blk * 128, 128)], sb[par],
                             sem[par])
            pltpu.async_copy(erest.at[pl.ds(blk * 384, 384)], eb[par],
                             sem[par])

    def wait1_issue2(par, i):
        @pl.when(vb(i))
        def _():
            pltpu.make_async_copy(esrc.at[pl.ds(0, 128)], sb[par],
                                  sem[par]).wait()
            pltpu.make_async_copy(erest.at[pl.ds(0, 384)], eb[par],
                                  sem[par]).wait()
            for jj in range(8):
                sl = pl.ds(jj * 16, 16)
                sb[par][sl] = sb[par][sl] + coff
            pltpu.async_copy(hcat.at[sb[par]], rows[par], sem[par])

    def compute(par, i):
        @pl.when(vb(i))
        def _():
            pltpu.make_async_copy(hcat.at[pl.ds(0, 128)], rows[par],
                                  sem[par]).wait()
            for jj in range(8):
                sl = pl.ds(jj * 16, 16)
                dstb[par][sl] = eb[par][sl]

            def mbody(g, _):
                g0 = g * 8
                lane0 = (g % 2) * 8
                n1c = _f32(eb[par][pl.ds(128 + (g // 2) * 16, 16)])
                for e in range(8):
                    k = g0 + e
                    n1 = n1c[jnp.full((16,), e, I32) + lane0]
                    for j in range(4):
                        slj = pl.ds(j * 16, 16)
                        slj1 = pl.ds(64 + j * 16, 16)
                        msg_a[k, slj] = rows[par][k, slj] * n1
                        msg_a[k, slj1] = rows[par][k, slj1] * n1
                return 0

            lax.fori_loop(0, 16, mbody, 0)
            pltpu.async_copy(msg_a, acc_as.at[dstb[par]], sem[par],
                             add=True).wait()

    issue1(0, 0)
    wait1_issue2(0, 0)
    issue1(1, 1)

    def body(i2, _):
        for par in (0, 1):
            i = i2 * 2 + par
            wait1_issue2(1 - par, i + 1)
            compute(par, i)
            issue1(par, i + 2)
        return 0

    lax.fori_loop(0, (NI + 2) // 2, body, 0)
    plsc.subcore_barrier()

    def _drain(off, sz):
        pltpu.sync_copy(acc_as.at[pl.ds(off, sz)], msg_a.at[pl.ds(0, sz)])
        pltpu.sync_copy(msg_a.at[pl.ds(0, sz)], acc_out_a.at[c, pl.ds(off, sz)])

    _rowshare(s, _drain)


@functools.partial(
    pl.kernel,
    out_type=jax.ShapeDtypeStruct((2, N, 128), F32),
    mesh=_mesh,
    scratch_types=[
        pltpu.VMEM((128,), I32),
        pltpu.VMEM((128,), I32),
        pltpu.VMEM((384,), I32),
        pltpu.VMEM((384,), I32),
        pltpu.VMEM((128,), I32),
        pltpu.VMEM((128,), I32),
        pltpu.VMEM((128, 128), F32),
        pltpu.VMEM((128, 128), F32),
        pltpu.VMEM((128, 128), F32),
        pltpu.SemaphoreType.DMA,
        pltpu.SemaphoreType.DMA,
        pltpu.VMEM_SHARED((N, 128), F32),
    ],
)
def k_msga(*refs):
    _k_msga(*refs)


def _k_msgb(hfull, esrc, erest, zeros_b, acc_out_b,
            sb0, sb1, eb0, eb1, dst0, dst1, n2buf, rows0, rows1, msg_b,
            sem0, sem1, acc_bs):
    c = lax.axis_index("c")
    s = lax.axis_index("s")
    sb = (sb0, sb1)
    eb = (eb0, eb1)
    dstb = (dst0, dst1)
    rows = (rows0, rows1)
    sem = (sem0, sem1)

    pltpu.sync_copy(zeros_b, msg_b)

    def _zero(off, sz):
        pltpu.sync_copy(msg_b.at[pl.ds(0, sz)], acc_bs.at[pl.ds(off, sz)])

    _rowshare(s, _zero)
    plsc.subcore_barrier()

    NI = (NBH + NSUB - 1) // NSUB

    def vb(i):
        return (i * NSUB + s) < NBH

    def issue1(par, i):
        blk = c * NBH + i * NSUB + s

        @pl.when(vb(i))
        def _():
            pltpu.async_copy(esrc.at[pl.ds(blk * 128, 128)], sb[par],
                             sem[par])
            pltpu.async_copy(erest.at[pl.ds(blk * 384, 384)], eb[par],
                             sem[par])

    def wait1_issue2(par, i):
        @pl.when(vb(i))
        def _():
            pltpu.make_async_copy(esrc.at[pl.ds(0, 128)], sb[par],
                                  sem[par]).wait()
            pltpu.make_async_copy(erest.at[pl.ds(0, 384)], eb[par],
                                  sem[par]).wait()
            pltpu.async_copy(hfull.at[sb[par]], rows[par], sem[par])

    def compute(par, i):
        @pl.when(vb(i))
        def _():
            pltpu.make_async_copy(hfull.at[pl.ds(0, 128)], rows[par],
                                  sem[par]).wait()
            for jj in range(8):
                sl = pl.ds(jj * 16, 16)
                dstb[par][sl] = eb[par][sl]
                e16 = _f32(eb[par][pl.ds(128 + jj * 16, 16)])
                m16 = _f32(eb[par][pl.ds(256 + jj * 16, 16)])
                n2buf[sl] = e16 * m16

            def mbody(g, _):
                g0 = g * 8
                lane0 = (g % 2) * 8
                n2c = n2buf[pl.ds((g // 2) * 16, 16)]
                for e in range(8):
                    k = g0 + e
                    n2 = n2c[jnp.full((16,), e, I32) + lane0]
                    for j in range(8):
                        slj = pl.ds(j * 16, 16)
                        msg_b[k, slj] = rows[par][k, slj] * n2
                return 0

            lax.fori_loop(0, 16, mbody, 0)
            pltpu.async_copy(msg_b, acc_bs.at[dstb[par]], sem[par],
                             add=True).wait()

    issue1(0, 0)
    wait1_issue2(0, 0)
    issue1(1, 1)

    def body(i2, _):
        for par in (0, 1):
            i = i2 * 2 + par
            wait1_issue2(1 - par, i + 1)
            compute(par, i)
            issue1(par, i + 2)
        return 0

    lax.fori_loop(0, (NI + 2) // 2, body, 0)
    plsc.subcore_barrier()

    def _drain(off, sz):
        pltpu.sync_copy(acc_bs.at[pl.ds(off, sz)], msg_b.at[pl.ds(0, sz)])
        pltpu.sync_copy(msg_b.at[pl.ds(0, sz)], acc_out_b.at[c, pl.ds(off, sz)])

    _rowshare(s, _drain)


@functools.partial(
    pl.kernel,
    out_type=jax.ShapeDtypeStruct((2, N, 128), F32),
    mesh=_mesh,
    scratch_types=[
        pltpu.VMEM((128,), I32),
        pltpu.VMEM((128,), I32),
        pltpu.VMEM((384,), I32),
        pltpu.VMEM((384,), I32),
        pltpu.VMEM((128,), I32),
        pltpu.VMEM((128,), I32),
        pltpu.VMEM((128,), F32),
        pltpu.VMEM((128, 128), F32),
        pltpu.VMEM((128, 128), F32),
        pltpu.VMEM((128, 128), F32),
        pltpu.SemaphoreType.DMA,
        pltpu.SemaphoreType.DMA,
        pltpu.VMEM_SHARED((N, 128), F32),
    ],
)
def k_msgb(*refs):
    _k_msgb(*refs)


def _k_out_body(acca_ref, accb_ref, dis_ref, n1_ref, n2_ref,
                z_ref, z1_ref, z2_ref):
    a0 = acca_ref[0]
    a1 = acca_ref[1]
    disv = dis_ref[0]
    dis2v = dis_ref[1]

    def halves(lo):
        return jnp.concatenate([a0[:, lo:lo + 64], a1[:, lo:lo + 64]], axis=1)

    z_ref[...] = jnp.maximum(halves(0) * disv, 0.0)
    z1_ref[...] = 0.9 * jnp.maximum(halves(64) * disv, 0.0) + 0.1 * n1_ref[...]
    zb = (accb_ref[0] + accb_ref[1]) * dis2v
    z2_ref[...] = 0.9 * jnp.maximum(zb, 0.0) + 0.1 * n2_ref[...]


def k_out(acc_a, acc_b, disr, noise1, noise2):
    o = jax.ShapeDtypeStruct((N, H), F32)
    return pl.pallas_call(
        _k_out_body,
        grid=(25,),
        in_specs=[
            pl.BlockSpec((2, 400, 128), lambda r: (0, r, 0)),
            pl.BlockSpec((2, 400, 128), lambda r: (0, r, 0)),
            pl.BlockSpec((2, 400, 1), lambda r: (0, r, 0)),
            pl.BlockSpec((400, 128), lambda r: (r, 0)),
            pl.BlockSpec((400, 128), lambda r: (r, 0)),
        ],
        out_specs=[
            pl.BlockSpec((400, 128), lambda r: (r, 0)),
            pl.BlockSpec((400, 128), lambda r: (r, 0)),
            pl.BlockSpec((400, 128), lambda r: (r, 0)),
        ],
        out_shape=(o, o, o),
    )(acc_a, acc_b, disr, noise1, noise2)


def kernel(x, edge_index, edge_weight, W):
    kf = jax.random.key(42)
    ka, kb, kn1, kn2 = jax.random.split(kf, 4)
    feat_mask = (jax.random.uniform(ka, (1, D)) > 0.2).astype(x.dtype)
    edge_mask = (jax.random.uniform(kb, (E,)) > 0.2).astype(edge_weight.dtype)
    noise1 = jax.random.normal(kn1, (N, H), dtype=x.dtype)
    noise2 = jax.random.normal(kn2, (N, H), dtype=x.dtype)

    W1 = W * feat_mask[0][:, None]
    wp = jnp.stack([
        jnp.concatenate([W[:, 0:64], W1[:, 0:64]], axis=1),
        jnp.concatenate([W[:, 64:128], W1[:, 64:128]], axis=1),
    ])

    src2d = edge_index[0].astype(I32).reshape(NB, 128)
    dst2d = edge_index[1].astype(I32).reshape(NB, 128)
    ewb = lax.bitcast_convert_type(edge_weight, I32).reshape(NB, 128)
    emb = lax.bitcast_convert_type(edge_mask, I32).reshape(NB, 128)
    esrc = src2d.reshape(NB * 128)
    erest = jnp.stack([dst2d, ewb, emb], axis=1).reshape(NB * 384)

    zeros1d = jnp.zeros((ROWS_A,), F32)
    zeros_a = jnp.zeros((128, 128), F32)

    dis_all = k_deg(erest, zeros1d)
    disr = dis_all.reshape(2, NP, 1)
    hcat = k_mat(x, wp, disr)
    hfull = k_mat2(x, W, disr)
    acc_a = k_msga(hcat, esrc, erest, zeros_a)
    acc_b = k_msgb(hfull, esrc, erest, zeros_a)
    z, z1, z2 = k_out(acc_a, acc_b, disr, noise1, noise2)
    return (z, z1, z2)
